# baseline (speedup 1.0000x reference)
import functools

import jax
import jax.numpy as jnp
from jax.experimental import pallas as pl
from jax.experimental.pallas import tpu as pltpu

BINS = 10
BLOCK_ROWS = 512
HIST_CHUNK_ROWS = 8
HIST_CHUNK_COLS = 256
LOSS_CHUNK_ROWS = 8


def _g10(x, t):
    ax = jnp.abs(x)
    e = jnp.exp(-ax)
    d = 1.0 / (1.0 + e)
    p = d * jnp.where(x >= 0.0, 1.0, e)
    return jnp.abs(p - t) * BINS, e


def _hist_kernel(x_ref, t_ref, ge_ref):
    i = pl.program_id(0)

    @pl.when(i == 0)
    def _init():
        for k in range(BINS):
            ge_ref[0, k] = 0

    rows, cols = x_ref.shape
    cr, cc = HIST_CHUNK_ROWS, HIST_CHUNK_COLS
    n_chunks = (rows // cr) * (cols // cc)
    cols_per = cols // cc

    def body(c, accs):
        r = (c // cols_per) * cr
        cc0 = (c % cols_per) * cc
        x = x_ref[pl.ds(r, cr), pl.ds(cc0, cc)]
        t = t_ref[pl.ds(r, cr), pl.ds(cc0, cc)]
        g10, _ = _g10(x, t)
        return tuple(
            acc + (g10 >= (k + 1)).astype(jnp.int32) for k, acc in enumerate(accs)
        )

    zero = jnp.zeros((cr, cc), jnp.int32)
    accs = jax.lax.fori_loop(0, n_chunks, body, (zero,) * (BINS - 1))
    for k in range(1, BINS):
        ge_ref[0, k] += jnp.sum(accs[k - 1])


def _loss_kernel(ge_ref, x_ref, t_ref, out_ref, *, tot):
    ge = [jnp.int32(tot)] + [ge_ref[0, k] for k in range(1, BINS)] + [jnp.int32(0)]
    counts = [ge[k] - ge[k + 1] for k in range(BINS)]
    n = functools.reduce(
        lambda a, b: a + b, [(c > 0).astype(jnp.int32) for c in counts]
    )
    nf = n.astype(jnp.float32)
    c = [jnp.where(cn > 0, 1.0 / (cn.astype(jnp.float32) * nf), 0.0) for cn in counts]

    rows, cols = x_ref.shape
    cr = LOSS_CHUNK_ROWS

    def body(i, _):
        r = i * cr
        x = x_ref[pl.ds(r, cr), :]
        t = t_ref[pl.ds(r, cr), :]
        g10, e = _g10(x, t)
        w_lo = jnp.where(
            g10 >= 2.0,
            jnp.where(g10 >= 3.0, jnp.where(g10 >= 4.0, c[4], c[3]), c[2]),
            jnp.where(g10 >= 1.0, c[1], c[0]),
        )
        w_hi = jnp.where(
            g10 >= 7.0,
            jnp.where(g10 >= 8.0, jnp.where(g10 >= 9.0, c[9], c[8]), c[7]),
            jnp.where(g10 >= 6.0, c[6], c[5]),
        )
        w = jnp.where(g10 >= 5.0, w_hi, w_lo)
        bce = jnp.maximum(x, 0.0) + jnp.log1p(e) - x * t
        out_ref[pl.ds(r, cr), :] = w * bce
        return 0

    jax.lax.fori_loop(0, rows // cr, body, 0)


def kernel(inputs, targets):
    rows, cols = inputs.shape
    tot = rows * cols
    grid = (rows // BLOCK_ROWS,)
    data_spec = pl.BlockSpec((BLOCK_ROWS, cols), lambda i: (i, 0))

    ge = pl.pallas_call(
        _hist_kernel,
        grid=grid,
        in_specs=[data_spec, data_spec],
        out_specs=pl.BlockSpec(memory_space=pltpu.SMEM),
        out_shape=jax.ShapeDtypeStruct((1, BINS), jnp.int32),
        compiler_params=pltpu.CompilerParams(
            dimension_semantics=("arbitrary",),
        ),
    )(inputs, targets)

    loss = pl.pallas_call(
        functools.partial(_loss_kernel, tot=tot),
        grid=grid,
        in_specs=[
            pl.BlockSpec(memory_space=pltpu.SMEM),
            data_spec,
            data_spec,
        ],
        out_specs=data_spec,
        out_shape=jax.ShapeDtypeStruct((rows, cols), jnp.float32),
        compiler_params=pltpu.CompilerParams(
            dimension_semantics=("parallel",),
        ),
    )(ge, inputs, targets)
    return loss


# speedup vs baseline: 1.4180x; 1.4180x over previous
import functools

import jax
import jax.numpy as jnp
from jax.experimental import pallas as pl
from jax.experimental.pallas import tpu as pltpu

BINS = 10
BLOCK_ROWS = 512


def _g10(x, t):
    ax = jnp.abs(x)
    e = jnp.exp(-ax)
    d = 1.0 / (1.0 + e)
    p = d * jnp.where(x >= 0.0, 1.0, e)
    return jnp.abs(p - t) * BINS, e


def _hist_kernel(x_ref, t_ref, ge_ref):
    i = pl.program_id(0)

    @pl.when(i == 0)
    def _init():
        for k in range(BINS):
            ge_ref[0, k] = 0

    g10, _ = _g10(x_ref[...], t_ref[...])
    for k in range(1, BINS):
        ge_ref[0, k] += jnp.sum((g10 >= k).astype(jnp.int32))


def _loss_kernel(ge_ref, x_ref, t_ref, out_ref, *, tot):
    ge = [jnp.int32(tot)] + [ge_ref[0, k] for k in range(1, BINS)] + [jnp.int32(0)]
    counts = [ge[k] - ge[k + 1] for k in range(BINS)]
    n = functools.reduce(
        lambda a, b: a + b, [(c > 0).astype(jnp.int32) for c in counts]
    )
    nf = n.astype(jnp.float32)
    c = [1.0 / (jnp.maximum(cn, 1).astype(jnp.float32) * nf) for cn in counts]

    x = x_ref[...]
    t = t_ref[...]
    g10, e = _g10(x, t)
    w_lo = jnp.where(
        g10 >= 2.0,
        jnp.where(g10 >= 3.0, jnp.where(g10 >= 4.0, c[4], c[3]), c[2]),
        jnp.where(g10 >= 1.0, c[1], c[0]),
    )
    w_hi = jnp.where(
        g10 >= 7.0,
        jnp.where(g10 >= 8.0, jnp.where(g10 >= 9.0, c[9], c[8]), c[7]),
        jnp.where(g10 >= 6.0, c[6], c[5]),
    )
    w = jnp.where(g10 >= 5.0, w_hi, w_lo)
    bce = jnp.maximum(x, 0.0) + jnp.log1p(e) - x * t
    out_ref[...] = w * bce


def kernel(inputs, targets):
    rows, cols = inputs.shape
    tot = rows * cols
    grid = (rows // BLOCK_ROWS,)
    data_spec = pl.BlockSpec((BLOCK_ROWS, cols), lambda i: (i, 0))

    ge = pl.pallas_call(
        _hist_kernel,
        grid=grid,
        in_specs=[data_spec, data_spec],
        out_specs=pl.BlockSpec(memory_space=pltpu.SMEM),
        out_shape=jax.ShapeDtypeStruct((1, BINS), jnp.int32),
        compiler_params=pltpu.CompilerParams(
            dimension_semantics=("arbitrary",),
        ),
    )(inputs, targets)

    loss = pl.pallas_call(
        functools.partial(_loss_kernel, tot=tot),
        grid=grid,
        in_specs=[
            pl.BlockSpec(memory_space=pltpu.SMEM),
            data_spec,
            data_spec,
        ],
        out_specs=data_spec,
        out_shape=jax.ShapeDtypeStruct((rows, cols), jnp.float32),
        compiler_params=pltpu.CompilerParams(
            dimension_semantics=("parallel",),
        ),
    )(ge, inputs, targets)
    return loss


# speedup vs baseline: 1.5415x; 1.0871x over previous
import functools

import jax
import jax.numpy as jnp
from jax.experimental import pallas as pl
from jax.experimental.pallas import tpu as pltpu

BINS = 10
BLOCK_ROWS = 512


def _g10(x, t):
    ax = jnp.abs(x)
    e = jnp.exp(-ax)
    d = 1.0 / (1.0 + e)
    p = d * jnp.where(x >= 0.0, 1.0, e)
    return jnp.abs(p - t) * BINS, d


def _hist_kernel(x_ref, t_ref, ge_ref):
    i = pl.program_id(0)

    @pl.when(i == 0)
    def _init():
        for k in range(BINS):
            ge_ref[0, k] = 0

    g10, _ = _g10(x_ref[...], t_ref[...])
    for k in range(1, BINS):
        ge_ref[0, k] += jnp.sum((g10 >= k).astype(jnp.int32))


def _loss_kernel(ge_ref, x_ref, t_ref, out_ref, *, tot):
    ge = [jnp.int32(tot)] + [ge_ref[0, k] for k in range(1, BINS)] + [jnp.int32(0)]
    counts = [ge[k] - ge[k + 1] for k in range(BINS)]
    n = functools.reduce(
        lambda a, b: a + b, [(c > 0).astype(jnp.int32) for c in counts]
    )
    nf = n.astype(jnp.float32)
    c = [1.0 / (jnp.maximum(cn, 1).astype(jnp.float32) * nf) for cn in counts]

    x = x_ref[...]
    t = t_ref[...]
    g10, d = _g10(x, t)
    w_lo = jnp.where(
        g10 >= 2.0,
        jnp.where(g10 >= 3.0, jnp.where(g10 >= 4.0, c[4], c[3]), c[2]),
        jnp.where(g10 >= 1.0, c[1], c[0]),
    )
    w_hi = jnp.where(
        g10 >= 7.0,
        jnp.where(g10 >= 8.0, jnp.where(g10 >= 9.0, c[9], c[8]), c[7]),
        jnp.where(g10 >= 6.0, c[6], c[5]),
    )
    w = jnp.where(g10 >= 5.0, w_hi, w_lo)
    bce = jnp.maximum(x, 0.0) - jnp.log(d) - x * t
    out_ref[...] = w * bce


def kernel(inputs, targets):
    rows, cols = inputs.shape
    tot = rows * cols
    grid = (rows // BLOCK_ROWS,)
    data_spec = pl.BlockSpec((BLOCK_ROWS, cols), lambda i: (i, 0))

    ge = pl.pallas_call(
        _hist_kernel,
        grid=grid,
        in_specs=[data_spec, data_spec],
        out_specs=pl.BlockSpec(memory_space=pltpu.SMEM),
        out_shape=jax.ShapeDtypeStruct((1, BINS), jnp.int32),
        compiler_params=pltpu.CompilerParams(
            dimension_semantics=("arbitrary",),
        ),
    )(inputs, targets)

    loss = pl.pallas_call(
        functools.partial(_loss_kernel, tot=tot),
        grid=grid,
        in_specs=[
            pl.BlockSpec(memory_space=pltpu.SMEM),
            data_spec,
            data_spec,
        ],
        out_specs=data_spec,
        out_shape=jax.ShapeDtypeStruct((rows, cols), jnp.float32),
        compiler_params=pltpu.CompilerParams(
            dimension_semantics=("parallel",),
        ),
    )(ge, inputs, targets)
    return loss


# speedup vs baseline: 1.5601x; 1.0120x over previous
import functools

import jax
import jax.numpy as jnp
from jax.experimental import pallas as pl
from jax.experimental.pallas import tpu as pltpu

BINS = 10
BLOCK_ROWS = 1024


def _g10(x, t):
    ax = jnp.abs(x)
    e = jnp.exp(-ax)
    d = 1.0 / (1.0 + e)
    p = d * jnp.where(x >= 0.0, 1.0, e)
    return jnp.abs(p - t) * BINS, d


def _hist_kernel(x_ref, t_ref, ge_ref):
    i = pl.program_id(0)

    @pl.when(i == 0)
    def _init():
        for k in range(BINS):
            ge_ref[0, k] = 0

    g10, _ = _g10(x_ref[...], t_ref[...])
    for k in range(1, BINS):
        ge_ref[0, k] += jnp.sum((g10 >= k).astype(jnp.int32))


def _loss_kernel(ge_ref, x_ref, t_ref, out_ref, *, tot):
    ge = [jnp.int32(tot)] + [ge_ref[0, k] for k in range(1, BINS)] + [jnp.int32(0)]
    counts = [ge[k] - ge[k + 1] for k in range(BINS)]
    n = functools.reduce(
        lambda a, b: a + b, [(c > 0).astype(jnp.int32) for c in counts]
    )
    nf = n.astype(jnp.float32)
    c = [1.0 / (jnp.maximum(cn, 1).astype(jnp.float32) * nf) for cn in counts]

    x = x_ref[...]
    t = t_ref[...]
    g10, d = _g10(x, t)
    w_lo = jnp.where(
        g10 >= 2.0,
        jnp.where(g10 >= 3.0, jnp.where(g10 >= 4.0, c[4], c[3]), c[2]),
        jnp.where(g10 >= 1.0, c[1], c[0]),
    )
    w_hi = jnp.where(
        g10 >= 7.0,
        jnp.where(g10 >= 8.0, jnp.where(g10 >= 9.0, c[9], c[8]), c[7]),
        jnp.where(g10 >= 6.0, c[6], c[5]),
    )
    w = jnp.where(g10 >= 5.0, w_hi, w_lo)
    bce = jnp.maximum(x, 0.0) - jnp.log(d) - x * t
    out_ref[...] = w * bce


def kernel(inputs, targets):
    rows, cols = inputs.shape
    tot = rows * cols
    grid = (rows // BLOCK_ROWS,)
    data_spec = pl.BlockSpec((BLOCK_ROWS, cols), lambda i: (i, 0))

    ge = pl.pallas_call(
        _hist_kernel,
        grid=grid,
        in_specs=[data_spec, data_spec],
        out_specs=pl.BlockSpec(memory_space=pltpu.SMEM),
        out_shape=jax.ShapeDtypeStruct((1, BINS), jnp.int32),
        compiler_params=pltpu.CompilerParams(
            dimension_semantics=("arbitrary",),
        ),
    )(inputs, targets)

    loss = pl.pallas_call(
        functools.partial(_loss_kernel, tot=tot),
        grid=grid,
        in_specs=[
            pl.BlockSpec(memory_space=pltpu.SMEM),
            data_spec,
            data_spec,
        ],
        out_specs=data_spec,
        out_shape=jax.ShapeDtypeStruct((rows, cols), jnp.float32),
        compiler_params=pltpu.CompilerParams(
            dimension_semantics=("parallel",),
        ),
    )(ge, inputs, targets)
    return loss


# speedup vs baseline: 1.6438x; 1.0537x over previous
import functools

import jax
import jax.numpy as jnp
from jax.experimental import pallas as pl
from jax.experimental.pallas import tpu as pltpu

BINS = 10
BLOCK_ROWS = 1024


def _g10(x, t):
    ax = jnp.abs(x)
    e = jnp.exp(-ax)
    d = 1.0 / (1.0 + e)
    p = d * jnp.where(x >= 0.0, 1.0, e)
    return jnp.abs(p - t) * BINS, d


def _hist_kernel(x_ref, t_ref, cnt_ref):
    i = pl.program_id(0)

    @pl.when(i == 0)
    def _init():
        for k in range(BINS):
            cnt_ref[0, k] = 0

    g10, _ = _g10(x_ref[...], t_ref[...])
    b = jnp.minimum(g10.astype(jnp.int32), BINS - 1)
    islo = b < 5
    sh6 = b * 6
    sh = jnp.where(islo, sh6, sh6 - 30)
    p = jnp.left_shift(jnp.int32(1), sh)
    plo = jnp.where(islo, p, 0)
    phi = p - plo
    for arr, base in ((plo, 0), (phi, 5)):
        s = arr
        for _ in range(5):
            h = s.shape[0] // 2
            s = s[:h] + s[h:]
        for f in range(5):
            cnt_ref[0, base + f] += jnp.sum((s >> (6 * f)) & 63)


def _loss_kernel(cnt_ref, x_ref, t_ref, out_ref, *, tot):
    del tot
    counts = [cnt_ref[0, k] for k in range(BINS)]
    n = functools.reduce(
        lambda a, b: a + b, [(c > 0).astype(jnp.int32) for c in counts]
    )
    nf = n.astype(jnp.float32)
    c = [1.0 / (jnp.maximum(cn, 1).astype(jnp.float32) * nf) for cn in counts]

    x = x_ref[...]
    t = t_ref[...]
    g10, d = _g10(x, t)
    w_lo = jnp.where(
        g10 >= 2.0,
        jnp.where(g10 >= 3.0, jnp.where(g10 >= 4.0, c[4], c[3]), c[2]),
        jnp.where(g10 >= 1.0, c[1], c[0]),
    )
    w_hi = jnp.where(
        g10 >= 7.0,
        jnp.where(g10 >= 8.0, jnp.where(g10 >= 9.0, c[9], c[8]), c[7]),
        jnp.where(g10 >= 6.0, c[6], c[5]),
    )
    w = jnp.where(g10 >= 5.0, w_hi, w_lo)
    bce = jnp.maximum(x, 0.0) - jnp.log(d) - x * t
    out_ref[...] = w * bce


def kernel(inputs, targets):
    rows, cols = inputs.shape
    tot = rows * cols
    grid = (rows // BLOCK_ROWS,)
    data_spec = pl.BlockSpec((BLOCK_ROWS, cols), lambda i: (i, 0))

    ge = pl.pallas_call(
        _hist_kernel,
        grid=grid,
        in_specs=[data_spec, data_spec],
        out_specs=pl.BlockSpec(memory_space=pltpu.SMEM),
        out_shape=jax.ShapeDtypeStruct((1, BINS), jnp.int32),
        compiler_params=pltpu.CompilerParams(
            dimension_semantics=("arbitrary",),
        ),
    )(inputs, targets)

    loss = pl.pallas_call(
        functools.partial(_loss_kernel, tot=tot),
        grid=grid,
        in_specs=[
            pl.BlockSpec(memory_space=pltpu.SMEM),
            data_spec,
            data_spec,
        ],
        out_specs=data_spec,
        out_shape=jax.ShapeDtypeStruct((rows, cols), jnp.float32),
        compiler_params=pltpu.CompilerParams(
            dimension_semantics=("parallel",),
        ),
    )(ge, inputs, targets)
    return loss


# speedup vs baseline: 1.7129x; 1.0421x over previous
import functools

import jax
import jax.numpy as jnp
from jax.experimental import pallas as pl
from jax.experimental.pallas import tpu as pltpu

BINS = 10
BLOCK_ROWS = 1024


def _g10(x, t):
    h = jnp.tanh(jnp.abs(x) * 0.5)
    d = 0.5 + 0.5 * h
    p = jnp.where(x >= 0.0, d, 1.0 - d)
    return jnp.abs(p - t) * BINS, d


def _hist_kernel(x_ref, t_ref, cnt_ref):
    i = pl.program_id(0)

    @pl.when(i == 0)
    def _init():
        for k in range(BINS):
            cnt_ref[0, k] = 0

    g10, _ = _g10(x_ref[...], t_ref[...])
    b = jnp.minimum(g10.astype(jnp.int32), BINS - 1)
    islo = b < 5
    sh6 = b * 6
    sh = jnp.where(islo, sh6, sh6 - 30)
    p = jnp.left_shift(jnp.int32(1), sh)
    plo = jnp.where(islo, p, 0)
    phi = p - plo
    for arr, base in ((plo, 0), (phi, 5)):
        s = arr
        for _ in range(5):
            h = s.shape[0] // 2
            s = s[:h] + s[h:]
        for f in range(5):
            cnt_ref[0, base + f] += jnp.sum((s >> (6 * f)) & 63)


def _loss_kernel(cnt_ref, x_ref, t_ref, out_ref, *, tot):
    del tot
    counts = [cnt_ref[0, k] for k in range(BINS)]
    n = functools.reduce(
        lambda a, b: a + b, [(c > 0).astype(jnp.int32) for c in counts]
    )
    nf = n.astype(jnp.float32)
    c = [1.0 / (jnp.maximum(cn, 1).astype(jnp.float32) * nf) for cn in counts]

    x = x_ref[...]
    t = t_ref[...]
    g10, d = _g10(x, t)
    w_lo = jnp.where(
        g10 >= 2.0,
        jnp.where(g10 >= 3.0, jnp.where(g10 >= 4.0, c[4], c[3]), c[2]),
        jnp.where(g10 >= 1.0, c[1], c[0]),
    )
    w_hi = jnp.where(
        g10 >= 7.0,
        jnp.where(g10 >= 8.0, jnp.where(g10 >= 9.0, c[9], c[8]), c[7]),
        jnp.where(g10 >= 6.0, c[6], c[5]),
    )
    w = jnp.where(g10 >= 5.0, w_hi, w_lo)
    bce = jnp.maximum(x, 0.0) - jnp.log(d) - x * t
    out_ref[...] = w * bce


def kernel(inputs, targets):
    rows, cols = inputs.shape
    tot = rows * cols
    grid = (rows // BLOCK_ROWS,)
    data_spec = pl.BlockSpec((BLOCK_ROWS, cols), lambda i: (i, 0))

    ge = pl.pallas_call(
        _hist_kernel,
        grid=grid,
        in_specs=[data_spec, data_spec],
        out_specs=pl.BlockSpec(memory_space=pltpu.SMEM),
        out_shape=jax.ShapeDtypeStruct((1, BINS), jnp.int32),
        compiler_params=pltpu.CompilerParams(
            dimension_semantics=("arbitrary",),
        ),
    )(inputs, targets)

    loss = pl.pallas_call(
        functools.partial(_loss_kernel, tot=tot),
        grid=grid,
        in_specs=[
            pl.BlockSpec(memory_space=pltpu.SMEM),
            data_spec,
            data_spec,
        ],
        out_specs=data_spec,
        out_shape=jax.ShapeDtypeStruct((rows, cols), jnp.float32),
        compiler_params=pltpu.CompilerParams(
            dimension_semantics=("parallel",),
        ),
    )(ge, inputs, targets)
    return loss
